# baseline (speedup 1.0000x reference)
import functools

import jax
import jax.numpy as jnp
from jax import lax
from jax.experimental import pallas as pl
from jax.experimental.pallas import tpu as pltpu, tpu_sc as plsc

VOCAB = 100000
D_MODEL = 128
BATCH = 4096
HIST = 200

_B = BATCH * HIST
_NC, _NS = 2, 16
_NW = _NC * _NS
_BPW = _B // _NW
_NBUF = 4
_LOOKAHEAD = 2
_CHUNK = 200
_NCHUNK = _BPW // _CHUNK
_NGRP = (_NCHUNK - 2 * _NBUF) // _NBUF

_mesh = plsc.VectorSubcoreMesh(core_axis_name="c", subcore_axis_name="s")


@functools.partial(
    pl.kernel,
    out_type=jax.ShapeDtypeStruct((_B, D_MODEL), jnp.float32),
    mesh=_mesh,
    scratch_types=[
        pltpu.VMEM((_BPW,), jnp.int32),
        *[pltpu.VMEM((_CHUNK, D_MODEL), jnp.float32) for _ in range(_NBUF)],
        *[pltpu.SemaphoreType.DMA for _ in range(2 * _NBUF)],
    ],
)
def _gather_kernel(idx_hbm, table_hbm, out_hbm, idx_v, *bufs):
    rows_v = bufs[:_NBUF]
    gsem = bufs[_NBUF:2 * _NBUF]
    ssem = bufs[2 * _NBUF:]

    wid = lax.axis_index("s") * _NC + lax.axis_index("c")
    base = wid * _BPW

    pltpu.sync_copy(idx_hbm.at[pl.ds(base, _BPW)], idx_v)

    def chunk_idx(i):
        return idx_v.at[pl.ds(i * _CHUNK, _CHUNK)]

    def gather_start(j, bj):
        pltpu.async_copy(table_hbm.at[chunk_idx(j)], rows_v[bj], gsem[bj])

    def gather_wait(i, b):
        pltpu.make_async_copy(
            table_hbm.at[chunk_idx(i)], rows_v[b], gsem[b]).wait()

    def store_start(i, b):
        pltpu.async_copy(rows_v[b], out_hbm.at[pl.ds(base + i * _CHUNK,
                                                     _CHUNK)], ssem[b])

    def store_wait(b):
        pltpu.make_async_copy(
            rows_v[b], out_hbm.at[pl.ds(base, _CHUNK)], ssem[b]).wait()

    def step(i, b, reissue, with_store_wait):
        gather_wait(i, b)
        store_start(i, b)
        if reissue:
            j = i + _LOOKAHEAD
            bj = (b + _LOOKAHEAD) % _NBUF
            if with_store_wait:
                store_wait(bj)
            gather_start(j, bj)

    for b in range(_LOOKAHEAD):
        gather_start(b, b)

    for i in range(_NBUF):
        step(i, i % _NBUF, reissue=True,
             with_store_wait=(i >= _NBUF - _LOOKAHEAD))

    def group(g, carry):
        i0 = _NBUF + g * _NBUF
        for b in range(_NBUF):
            step(i0 + b, b, reissue=True, with_store_wait=True)
        return carry

    lax.fori_loop(0, _NGRP, group, 0)

    i0 = _NBUF + _NGRP * _NBUF
    for k in range(_NBUF):
        i = i0 + k
        step(i, i % _NBUF, reissue=(i + _LOOKAHEAD < _NCHUNK),
             with_store_wait=True)

    for b in range(_NBUF):
        store_wait(b)


def kernel(x, table):
    idx = x.reshape(_B).astype(jnp.int32)
    out = _gather_kernel(idx, table)
    return out.reshape(BATCH, HIST, D_MODEL)
